# initial kernel scaffold (unmeasured)
import jax
import jax.numpy as jnp
from jax import lax
from jax.experimental import pallas as pl
from jax.experimental.pallas import tpu as pltpu

N_DEV = 4
V_PER = 8192
T = 4096
D = 2048
T_CHUNK = T // N_DEV
N_HOPS = 2 * (N_DEV - 1)


def kernel(ids, E):
    my = lax.axis_index("i")
    offset = my * V_PER
    local = jnp.clip(ids - offset, 0, V_PER - 1)
    mask = (ids >= offset) & (ids < offset + V_PER)
    partial = jnp.where(mask[:, None], jnp.take(E, local, axis=0), 0.0)
    partial = partial.astype(jnp.bfloat16)

    def body(x_ref, out_ref, comm_ref, send_sems, recv_sems):
        my_pos = lax.axis_index("i")
        left = jnp.mod(my_pos - 1, N_DEV)
        right = jnp.mod(my_pos + 1, N_DEV)

        barrier_sem = pltpu.get_barrier_semaphore()
        for nbr in (left, right):
            pl.semaphore_signal(
                barrier_sem, inc=1,
                device_id=(nbr,), device_id_type=pl.DeviceIdType.MESH,
            )
        pl.semaphore_wait(barrier_sem, 2)

        def rows(c):
            return pl.ds(c * T_CHUNK, T_CHUNK)

        comm_ref[0, :, :] = x_ref[rows(my_pos), :]

        for h in range(N_HOPS):
            rdma = pltpu.make_async_remote_copy(
                src_ref=comm_ref.at[h],
                dst_ref=comm_ref.at[h + 1],
                send_sem=send_sems.at[h],
                recv_sem=recv_sems.at[h],
                device_id=(right,),
                device_id_type=pl.DeviceIdType.MESH,
            )
            rdma.start()
            rdma.wait()

            if h < N_DEV - 1:
                rc = jnp.mod(my_pos - h - 1, N_DEV)
                comm_ref[h + 1, :, :] = comm_ref[h + 1, :, :] + x_ref[rows(rc), :]
                if h == N_DEV - 2:
                    out_ref[rows(right), :] = comm_ref[h + 1, :, :].astype(
                        jnp.float32
                    )
            else:
                oc = jnp.mod(my_pos - (h - (N_DEV - 1)), N_DEV)
                out_ref[rows(oc), :] = comm_ref[h + 1, :, :].astype(jnp.float32)

    out = pl.pallas_call(
        body,
        out_shape=jax.ShapeDtypeStruct((T, D), jnp.float32),
        in_specs=[pl.BlockSpec(memory_space=pltpu.VMEM)],
        out_specs=pl.BlockSpec(memory_space=pltpu.VMEM),
        scratch_shapes=[
            pltpu.VMEM((N_HOPS + 1, T_CHUNK, D), jnp.bfloat16),
            pltpu.SemaphoreType.DMA((N_HOPS,)),
            pltpu.SemaphoreType.DMA((N_HOPS,)),
        ],
        compiler_params=pltpu.CompilerParams(
            collective_id=0,
            vmem_limit_bytes=128 * 1024 * 1024,
        ),
    )(partial)
    return out


# baseline (device time: 604893 ns/iter reference)
import jax
import jax.numpy as jnp
from jax import lax
from jax.experimental import pallas as pl
from jax.experimental.pallas import tpu as pltpu

N_DEV = 4
V_PER = 8192
T = 4096
D = 2048
T_CHUNK = T // N_DEV
N_HOPS = 2 * (N_DEV - 1)


def kernel(ids, E):
    my = lax.axis_index("i")
    offset = my * V_PER
    local = jnp.clip(ids - offset, 0, V_PER - 1)
    mask = (ids >= offset) & (ids < offset + V_PER)
    partial = jnp.where(mask[:, None], jnp.take(E, local, axis=0), 0.0)
    partial = partial.astype(jnp.bfloat16)

    def body(x_ref, out_ref, comm_ref, send_sems, recv_sems):
        my_pos = lax.axis_index("i")
        left = jnp.mod(my_pos - 1, N_DEV)
        right = jnp.mod(my_pos + 1, N_DEV)

        barrier_sem = pltpu.get_barrier_semaphore()
        for nbr in (left, right):
            pl.semaphore_signal(
                barrier_sem, inc=1,
                device_id=(nbr,), device_id_type=pl.DeviceIdType.MESH,
            )
        pl.semaphore_wait(barrier_sem, 2)

        def rows(c):
            return pl.ds(c * T_CHUNK, T_CHUNK)

        comm_ref[0, :, :] = x_ref[rows(my_pos), :]

        for h in range(N_HOPS):
            rdma = pltpu.make_async_remote_copy(
                src_ref=comm_ref.at[h],
                dst_ref=comm_ref.at[h + 1],
                send_sem=send_sems.at[h],
                recv_sem=recv_sems.at[h],
                device_id=(right,),
                device_id_type=pl.DeviceIdType.MESH,
            )
            rdma.start()
            rdma.wait()

            if h < N_DEV - 1:
                rc = jnp.mod(my_pos - h - 1, N_DEV)
                comm_ref[h + 1, :, :] = comm_ref[h + 1, :, :] + x_ref[rows(rc), :]
                if h == N_DEV - 2:
                    out_ref[rows(right), :] = comm_ref[h + 1, :, :]
            else:
                oc = jnp.mod(my_pos - (h - (N_DEV - 1)), N_DEV)
                out_ref[rows(oc), :] = comm_ref[h + 1, :, :]

    out = pl.pallas_call(
        body,
        out_shape=jax.ShapeDtypeStruct((T, D), jnp.bfloat16),
        in_specs=[pl.BlockSpec(memory_space=pltpu.VMEM)],
        out_specs=pl.BlockSpec(memory_space=pltpu.VMEM),
        scratch_shapes=[
            pltpu.VMEM((N_HOPS + 1, T_CHUNK, D), jnp.bfloat16),
            pltpu.SemaphoreType.DMA((N_HOPS,)),
            pltpu.SemaphoreType.DMA((N_HOPS,)),
        ],
        compiler_params=pltpu.CompilerParams(
            collective_id=0,
            vmem_limit_bytes=128 * 1024 * 1024,
        ),
    )(partial)
    return out.astype(jnp.float32)


# device time: 470536 ns/iter; 1.2855x vs baseline; 1.2855x over previous
import jax
import jax.numpy as jnp
from jax import lax
from jax.experimental import pallas as pl
from jax.experimental.pallas import tpu as pltpu

N_DEV = 4
V_PER = 8192
T = 4096
D = 2048
T_CHUNK = T // N_DEV
HALF = D // 2
N_HOPS = 2 * (N_DEV - 1)


def kernel(ids, E):
    my = lax.axis_index("i")
    offset = my * V_PER
    local = jnp.clip(ids - offset, 0, V_PER - 1)
    mask = (ids >= offset) & (ids < offset + V_PER)
    partial = jnp.where(mask[:, None], jnp.take(E, local, axis=0), 0.0)
    partial = partial.astype(jnp.bfloat16)

    def body(x_ref, out_ref, comm_ref, send_sems, recv_sems):
        my_pos = lax.axis_index("i")
        left = jnp.mod(my_pos - 1, N_DEV)
        right = jnp.mod(my_pos + 1, N_DEV)

        barrier_sem = pltpu.get_barrier_semaphore()
        for nbr in (left, right):
            pl.semaphore_signal(
                barrier_sem, inc=1,
                device_id=(nbr,), device_id_type=pl.DeviceIdType.MESH,
            )
        pl.semaphore_wait(barrier_sem, 2)

        def rows(c):
            return pl.ds(c * T_CHUNK, T_CHUNK)

        R, L = 0, 1
        COLS = {R: slice(0, HALF), L: slice(HALF, D)}

        comm_ref[R, 0, :, :] = x_ref[rows(my_pos), COLS[R]]
        comm_ref[L, 0, :, :] = x_ref[rows(my_pos), COLS[L]]

        for h in range(N_HOPS):
            rdmas = []
            for r, nbr in ((R, right), (L, left)):
                rdma = pltpu.make_async_remote_copy(
                    src_ref=comm_ref.at[r, h],
                    dst_ref=comm_ref.at[r, h + 1],
                    send_sem=send_sems.at[r, h],
                    recv_sem=recv_sems.at[r, h],
                    device_id=(nbr,),
                    device_id_type=pl.DeviceIdType.MESH,
                )
                rdma.start()
                rdmas.append(rdma)
            for rdma in rdmas:
                rdma.wait()

            if h < N_DEV - 1:
                rc_r = jnp.mod(my_pos - h - 1, N_DEV)
                rc_l = jnp.mod(my_pos + h + 1, N_DEV)
                comm_ref[R, h + 1, :, :] = (
                    comm_ref[R, h + 1, :, :] + x_ref[rows(rc_r), COLS[R]]
                )
                comm_ref[L, h + 1, :, :] = (
                    comm_ref[L, h + 1, :, :] + x_ref[rows(rc_l), COLS[L]]
                )
                if h == N_DEV - 2:
                    out_ref[rows(right), COLS[R]] = comm_ref[R, h + 1, :, :]
                    out_ref[rows(left), COLS[L]] = comm_ref[L, h + 1, :, :]
            else:
                oc_r = jnp.mod(my_pos - (h - (N_DEV - 1)), N_DEV)
                oc_l = jnp.mod(my_pos + (h - (N_DEV - 1)), N_DEV)
                out_ref[rows(oc_r), COLS[R]] = comm_ref[R, h + 1, :, :]
                out_ref[rows(oc_l), COLS[L]] = comm_ref[L, h + 1, :, :]

    out = pl.pallas_call(
        body,
        out_shape=jax.ShapeDtypeStruct((T, D), jnp.bfloat16),
        in_specs=[pl.BlockSpec(memory_space=pltpu.VMEM)],
        out_specs=pl.BlockSpec(memory_space=pltpu.VMEM),
        scratch_shapes=[
            pltpu.VMEM((2, N_HOPS + 1, T_CHUNK, HALF), jnp.bfloat16),
            pltpu.SemaphoreType.DMA((2, N_HOPS)),
            pltpu.SemaphoreType.DMA((2, N_HOPS)),
        ],
        compiler_params=pltpu.CompilerParams(
            collective_id=0,
            vmem_limit_bytes=128 * 1024 * 1024,
        ),
    )(partial)
    return out.astype(jnp.float32)


# device time: 364182 ns/iter; 1.6610x vs baseline; 1.2920x over previous
import jax
import jax.numpy as jnp
from jax import lax
from jax.experimental import pallas as pl
from jax.experimental.pallas import tpu as pltpu

N_DEV = 4
V_PER = 8192
T = 4096
D = 2048
T_CHUNK = T // N_DEV
HALF = D // 2
N_HOPS = 2 * (N_DEV - 1)
UNROLL = 8


def kernel(ids, E):
    my = lax.axis_index("i")
    offset = my * V_PER
    local = jnp.clip(ids - offset, 0, V_PER - 1).astype(jnp.int32)
    mask = ((ids >= offset) & (ids < offset + V_PER)).astype(jnp.bfloat16)
    mask = mask[:, None]

    def body(ids_ref, mask_ref, e_ref, out_ref, comm_ref, stage_ref,
             send_sems, recv_sems, gsem):
        my_pos = lax.axis_index("i")
        left = jnp.mod(my_pos - 1, N_DEV)
        right = jnp.mod(my_pos + 1, N_DEV)

        barrier_sem = pltpu.get_barrier_semaphore()
        for nbr in (left, right):
            pl.semaphore_signal(
                barrier_sem, inc=1,
                device_id=(nbr,), device_id_type=pl.DeviceIdType.MESH,
            )
        pl.semaphore_wait(barrier_sem, 2)

        def rows(c):
            return pl.ds(c * T_CHUNK, T_CHUNK)

        def issue_gather(c, slot):
            base = c * T_CHUNK

            def issue(t, carry):
                for u in range(UNROLL):
                    row = ids_ref[base + t * UNROLL + u]
                    pltpu.make_async_copy(
                        e_ref.at[row], stage_ref.at[slot, t * UNROLL + u], gsem
                    ).start()
                return carry

            lax.fori_loop(0, T_CHUNK // UNROLL, issue, 0)

        def drain_gather(c, slot):
            base = c * T_CHUNK

            def drain(t, carry):
                for u in range(UNROLL):
                    row = ids_ref[base + t * UNROLL + u]
                    pltpu.make_async_copy(
                        e_ref.at[row], stage_ref.at[slot, t * UNROLL + u], gsem
                    ).wait()
                return carry

            lax.fori_loop(0, T_CHUNK // UNROLL, drain, 0)
            out_ref[rows(c), :] = (
                stage_ref[slot, :, :].astype(jnp.bfloat16) * mask_ref[rows(c), :]
            )

        R, L = 0, 1
        COLS = {R: slice(0, HALF), L: slice(HALF, D)}

        issue_gather(my_pos, 0)
        drain_gather(my_pos, 0)
        comm_ref[R, 0, :, :] = out_ref[rows(my_pos), COLS[R]]
        comm_ref[L, 0, :, :] = out_ref[rows(my_pos), COLS[L]]

        for h in range(N_HOPS):
            rdmas = []
            for r, nbr in ((R, right), (L, left)):
                rdma = pltpu.make_async_remote_copy(
                    src_ref=comm_ref.at[r, h],
                    dst_ref=comm_ref.at[r, h + 1],
                    send_sem=send_sems.at[r, h],
                    recv_sem=recv_sems.at[r, h],
                    device_id=(nbr,),
                    device_id_type=pl.DeviceIdType.MESH,
                )
                rdma.start()
                rdmas.append(rdma)

            if h == 0:
                issue_gather(right, 1)
                drain_gather(right, 1)
                issue_gather(left, 0)
                drain_gather(left, 0)
            elif h == 1:
                opp = jnp.mod(my_pos + 2, N_DEV)
                issue_gather(opp, 1)
                drain_gather(opp, 1)

            for rdma in rdmas:
                rdma.wait()

            if h < N_DEV - 1:
                rc_r = jnp.mod(my_pos - h - 1, N_DEV)
                rc_l = jnp.mod(my_pos + h + 1, N_DEV)
                comm_ref[R, h + 1, :, :] = (
                    comm_ref[R, h + 1, :, :] + out_ref[rows(rc_r), COLS[R]]
                )
                comm_ref[L, h + 1, :, :] = (
                    comm_ref[L, h + 1, :, :] + out_ref[rows(rc_l), COLS[L]]
                )
                if h == N_DEV - 2:
                    out_ref[rows(right), COLS[R]] = comm_ref[R, h + 1, :, :]
                    out_ref[rows(left), COLS[L]] = comm_ref[L, h + 1, :, :]
            else:
                oc_r = jnp.mod(my_pos - (h - (N_DEV - 1)), N_DEV)
                oc_l = jnp.mod(my_pos + (h - (N_DEV - 1)), N_DEV)
                out_ref[rows(oc_r), COLS[R]] = comm_ref[R, h + 1, :, :]
                out_ref[rows(oc_l), COLS[L]] = comm_ref[L, h + 1, :, :]

    out = pl.pallas_call(
        body,
        out_shape=jax.ShapeDtypeStruct((T, D), jnp.bfloat16),
        in_specs=[
            pl.BlockSpec(memory_space=pltpu.SMEM),
            pl.BlockSpec(memory_space=pltpu.VMEM),
            pl.BlockSpec(memory_space=pl.ANY),
        ],
        out_specs=pl.BlockSpec(memory_space=pltpu.VMEM),
        scratch_shapes=[
            pltpu.VMEM((2, N_HOPS + 1, T_CHUNK, HALF), jnp.bfloat16),
            pltpu.VMEM((2, T_CHUNK, D), jnp.float32),
            pltpu.SemaphoreType.DMA((2, N_HOPS)),
            pltpu.SemaphoreType.DMA((2, N_HOPS)),
            pltpu.SemaphoreType.DMA,
        ],
        compiler_params=pltpu.CompilerParams(
            collective_id=0,
            vmem_limit_bytes=128 * 1024 * 1024,
        ),
    )(local, mask, E)
    return out.astype(jnp.float32)


# device time: 362308 ns/iter; 1.6696x vs baseline; 1.0052x over previous
import jax
import jax.numpy as jnp
from jax import lax
from jax.experimental import pallas as pl
from jax.experimental.pallas import tpu as pltpu

N_DEV = 4
V_PER = 8192
T = 4096
D = 2048
T_CHUNK = T // N_DEV
HALF = D // 2
N_HOPS = 2 * (N_DEV - 1)
UNROLL = 8


def kernel(ids, E):
    my = lax.axis_index("i")
    offset = my * V_PER
    local = jnp.clip(ids - offset, 0, V_PER - 1).astype(jnp.int32)
    mask = ((ids >= offset) & (ids < offset + V_PER)).astype(jnp.bfloat16)
    mask = mask[:, None]

    def body(ids_ref, mask_ref, e_ref, out_ref, comm_ref, stage_ref,
             send_sems, recv_sems, gsems):
        my_pos = lax.axis_index("i")
        left = jnp.mod(my_pos - 1, N_DEV)
        right = jnp.mod(my_pos + 1, N_DEV)
        opp = jnp.mod(my_pos + 2, N_DEV)

        barrier_sem = pltpu.get_barrier_semaphore()
        for nbr in (left, right):
            pl.semaphore_signal(
                barrier_sem, inc=1,
                device_id=(nbr,), device_id_type=pl.DeviceIdType.MESH,
            )
        pl.semaphore_wait(barrier_sem, 2)

        def rows(c):
            return pl.ds(c * T_CHUNK, T_CHUNK)

        def issue_gather(c, slot, s):
            base = c * T_CHUNK

            def issue(t, carry):
                for u in range(UNROLL):
                    row = ids_ref[base + t * UNROLL + u]
                    pltpu.make_async_copy(
                        e_ref.at[row], stage_ref.at[slot, t * UNROLL + u],
                        gsems.at[s],
                    ).start()
                return carry

            lax.fori_loop(0, T_CHUNK // UNROLL, issue, 0)

        def drain_gather(c, slot, s):
            base = c * T_CHUNK

            def drain(t, carry):
                for u in range(UNROLL):
                    row = ids_ref[base + t * UNROLL + u]
                    pltpu.make_async_copy(
                        e_ref.at[row], stage_ref.at[slot, t * UNROLL + u],
                        gsems.at[s],
                    ).wait()
                return carry

            lax.fori_loop(0, T_CHUNK // UNROLL, drain, 0)
            out_ref[rows(c), :] = (
                stage_ref[slot, :, :].astype(jnp.bfloat16) * mask_ref[rows(c), :]
            )

        R, L = 0, 1
        COLS = {R: slice(0, HALF), L: slice(HALF, D)}

        issue_gather(my_pos, 0, 0)
        drain_gather(my_pos, 0, 0)
        comm_ref[R, 0, :, :] = out_ref[rows(my_pos), COLS[R]]
        comm_ref[L, 0, :, :] = out_ref[rows(my_pos), COLS[L]]

        for h in range(N_HOPS):
            for r, nbr in ((R, right), (L, left)):
                rdma = pltpu.make_async_remote_copy(
                    src_ref=comm_ref.at[r, h],
                    dst_ref=comm_ref.at[r, h + 1],
                    send_sem=send_sems.at[r, h],
                    recv_sem=recv_sems.at[r, h],
                    device_id=(nbr,),
                    device_id_type=pl.DeviceIdType.MESH,
                )
                rdma.start()

            if h == N_DEV - 1:
                out_ref[rows(right), COLS[R]] = comm_ref[R, h, :, :]
                out_ref[rows(left), COLS[L]] = comm_ref[L, h, :, :]
            elif h > N_DEV - 1:
                oc_r = jnp.mod(my_pos - (h - N_DEV), N_DEV)
                oc_l = jnp.mod(my_pos + (h - N_DEV), N_DEV)
                out_ref[rows(oc_r), COLS[R]] = comm_ref[R, h, :, :]
                out_ref[rows(oc_l), COLS[L]] = comm_ref[L, h, :, :]

            if h == 0:
                issue_gather(right, 1, 1)
                issue_gather(left, 0, 2)
                drain_gather(right, 1, 1)
                drain_gather(left, 0, 2)
            elif h == 1:
                issue_gather(opp, 1, 3)
                drain_gather(opp, 1, 3)

            for r, nbr in ((R, right), (L, left)):
                pltpu.make_async_remote_copy(
                    src_ref=comm_ref.at[r, h],
                    dst_ref=comm_ref.at[r, h + 1],
                    send_sem=send_sems.at[r, h],
                    recv_sem=recv_sems.at[r, h],
                    device_id=(nbr,),
                    device_id_type=pl.DeviceIdType.MESH,
                ).wait()

            if h < N_DEV - 1:
                rc_r = jnp.mod(my_pos - h - 1, N_DEV)
                rc_l = jnp.mod(my_pos + h + 1, N_DEV)
                comm_ref[R, h + 1, :, :] = (
                    comm_ref[R, h + 1, :, :] + out_ref[rows(rc_r), COLS[R]]
                )
                comm_ref[L, h + 1, :, :] = (
                    comm_ref[L, h + 1, :, :] + out_ref[rows(rc_l), COLS[L]]
                )

        out_ref[rows(opp), COLS[R]] = comm_ref[R, N_HOPS, :, :]
        out_ref[rows(opp), COLS[L]] = comm_ref[L, N_HOPS, :, :]

    out = pl.pallas_call(
        body,
        out_shape=jax.ShapeDtypeStruct((T, D), jnp.bfloat16),
        in_specs=[
            pl.BlockSpec(memory_space=pltpu.SMEM),
            pl.BlockSpec(memory_space=pltpu.VMEM),
            pl.BlockSpec(memory_space=pl.ANY),
        ],
        out_specs=pl.BlockSpec(memory_space=pltpu.VMEM),
        scratch_shapes=[
            pltpu.VMEM((2, N_HOPS + 1, T_CHUNK, HALF), jnp.bfloat16),
            pltpu.VMEM((2, T_CHUNK, D), jnp.float32),
            pltpu.SemaphoreType.DMA((2, N_HOPS)),
            pltpu.SemaphoreType.DMA((2, N_HOPS)),
            pltpu.SemaphoreType.DMA((4,)),
        ],
        compiler_params=pltpu.CompilerParams(
            collective_id=0,
            vmem_limit_bytes=128 * 1024 * 1024,
        ),
    )(local, mask, E)
    return out.astype(jnp.float32)


# device time: 189301 ns/iter; 3.1954x vs baseline; 1.9139x over previous
import jax
import jax.numpy as jnp
from jax import lax
from jax.experimental import pallas as pl
from jax.experimental.pallas import tpu as pltpu

N_DEV = 4
V_PER = 8192
T = 4096
D = 2048
T_CHUNK = T // N_DEV
HALF = D // 2
N_HOPS = 2 * (N_DEV - 1)
UNROLL = 8


def kernel(ids, E):
    my = lax.axis_index("i")
    offset = my * V_PER
    owned = (ids >= offset) & (ids < offset + V_PER)
    local = jnp.where(owned, ids - offset, jnp.arange(T, dtype=jnp.int32))
    local = local.astype(jnp.int32)
    mask = owned.astype(jnp.bfloat16)[:, None]

    def body(ids_ref, mask_ref, e_ref, out_ref, comm_ref, stage_ref,
             send_sems, recv_sems, gsems):
        my_pos = lax.axis_index("i")
        left = jnp.mod(my_pos - 1, N_DEV)
        right = jnp.mod(my_pos + 1, N_DEV)
        opp = jnp.mod(my_pos + 2, N_DEV)

        barrier_sem = pltpu.get_barrier_semaphore()
        for nbr in (left, right):
            pl.semaphore_signal(
                barrier_sem, inc=1,
                device_id=(nbr,), device_id_type=pl.DeviceIdType.MESH,
            )
        pl.semaphore_wait(barrier_sem, 2)

        def rows(c):
            return pl.ds(c * T_CHUNK, T_CHUNK)

        def issue_gather(c, slot, s):
            base = c * T_CHUNK

            def issue(t, carry):
                for u in range(UNROLL):
                    row = ids_ref[base + t * UNROLL + u]
                    pltpu.make_async_copy(
                        e_ref.at[row], stage_ref.at[slot, t * UNROLL + u],
                        gsems.at[s],
                    ).start()
                return carry

            lax.fori_loop(0, T_CHUNK // UNROLL, issue, 0)

        def drain_gather(c, slot, s):
            base = c * T_CHUNK

            def drain(t, carry):
                for u in range(UNROLL):
                    row = ids_ref[base + t * UNROLL + u]
                    pltpu.make_async_copy(
                        e_ref.at[row], stage_ref.at[slot, t * UNROLL + u],
                        gsems.at[s],
                    ).wait()
                return carry

            lax.fori_loop(0, T_CHUNK // UNROLL, drain, 0)
            out_ref[rows(c), :] = (
                stage_ref[slot, :, :].astype(jnp.bfloat16) * mask_ref[rows(c), :]
            )

        R, L = 0, 1
        COLS = {R: slice(0, HALF), L: slice(HALF, D)}
        SUB = T_CHUNK // 2

        def sub(s):
            return pl.ds(s * SUB, SUB)

        def subrows(c, s):
            return pl.ds(c * T_CHUNK + s * SUB, SUB)

        def hop_rdma(r, h, s):
            nbr = right if r == R else left
            return pltpu.make_async_remote_copy(
                src_ref=comm_ref.at[r, h, sub(s)],
                dst_ref=comm_ref.at[r, h + 1, sub(s)],
                send_sem=send_sems.at[r, h, s],
                recv_sem=recv_sems.at[r, h, s],
                device_id=(nbr,),
                device_id_type=pl.DeviceIdType.MESH,
            )

        seed_base = my_pos * T_CHUNK
        for s, sem_i in ((0, 0), (1, 4)):

            def seed_issue(t, carry):
                for u in range(UNROLL):
                    row = ids_ref[seed_base + s * SUB + t * UNROLL + u]
                    pltpu.make_async_copy(
                        e_ref.at[row],
                        stage_ref.at[0, s * SUB + t * UNROLL + u],
                        gsems.at[sem_i],
                    ).start()
                return carry

            lax.fori_loop(0, SUB // UNROLL, seed_issue, 0)
        for s, sem_i in ((0, 0), (1, 4)):

            def seed_drain(t, carry):
                for u in range(UNROLL):
                    row = ids_ref[seed_base + s * SUB + t * UNROLL + u]
                    pltpu.make_async_copy(
                        e_ref.at[row],
                        stage_ref.at[0, s * SUB + t * UNROLL + u],
                        gsems.at[sem_i],
                    ).wait()
                return carry

            lax.fori_loop(0, SUB // UNROLL, seed_drain, 0)
            mcol = mask_ref[subrows(my_pos, s), :]
            comm_ref[R, 0, sub(s), :] = (
                stage_ref[0, sub(s), COLS[R]].astype(jnp.bfloat16) * mcol
            )
            comm_ref[L, 0, sub(s), :] = (
                stage_ref[0, sub(s), COLS[L]].astype(jnp.bfloat16) * mcol
            )
            for r in (R, L):
                hop_rdma(r, 0, s).start()

        for h in range(N_HOPS):
            if h == N_DEV - 1:
                out_ref[rows(right), COLS[R]] = comm_ref[R, h, :, :]
                out_ref[rows(left), COLS[L]] = comm_ref[L, h, :, :]
            elif h > N_DEV - 1:
                oc_r = jnp.mod(my_pos - (h - N_DEV), N_DEV)
                oc_l = jnp.mod(my_pos + (h - N_DEV), N_DEV)
                out_ref[rows(oc_r), COLS[R]] = comm_ref[R, h, :, :]
                out_ref[rows(oc_l), COLS[L]] = comm_ref[L, h, :, :]

            if h == 0:
                issue_gather(right, 1, 1)
                issue_gather(left, 0, 2)
                drain_gather(right, 1, 1)
                drain_gather(left, 0, 2)
            elif h == 1:
                issue_gather(opp, 1, 3)
                drain_gather(opp, 1, 3)

            rc_r = jnp.mod(my_pos - h - 1, N_DEV)
            rc_l = jnp.mod(my_pos + h + 1, N_DEV)
            for s in range(2):
                for r in (R, L):
                    hop_rdma(r, h, s).wait()
                if h < N_DEV - 1:
                    comm_ref[R, h + 1, sub(s), :] = (
                        comm_ref[R, h + 1, sub(s), :]
                        + out_ref[subrows(rc_r, s), COLS[R]]
                    )
                    comm_ref[L, h + 1, sub(s), :] = (
                        comm_ref[L, h + 1, sub(s), :]
                        + out_ref[subrows(rc_l, s), COLS[L]]
                    )
                if h < N_HOPS - 1:
                    for r in (R, L):
                        hop_rdma(r, h + 1, s).start()

        out_ref[rows(opp), COLS[R]] = comm_ref[R, N_HOPS, :, :]
        out_ref[rows(opp), COLS[L]] = comm_ref[L, N_HOPS, :, :]

    out = pl.pallas_call(
        body,
        out_shape=jax.ShapeDtypeStruct((T, D), jnp.bfloat16),
        in_specs=[
            pl.BlockSpec(memory_space=pltpu.SMEM),
            pl.BlockSpec(memory_space=pltpu.VMEM),
            pl.BlockSpec(memory_space=pl.ANY),
        ],
        out_specs=pl.BlockSpec(memory_space=pltpu.VMEM),
        scratch_shapes=[
            pltpu.VMEM((2, N_HOPS + 1, T_CHUNK, HALF), jnp.bfloat16),
            pltpu.VMEM((2, T_CHUNK, D), jnp.float32),
            pltpu.SemaphoreType.DMA((2, N_HOPS, 2)),
            pltpu.SemaphoreType.DMA((2, N_HOPS, 2)),
            pltpu.SemaphoreType.DMA((5,)),
        ],
        compiler_params=pltpu.CompilerParams(
            collective_id=0,
            vmem_limit_bytes=128 * 1024 * 1024,
        ),
    )(local, mask, E)
    return out
